# baseline (speedup 1.0000x reference)
import functools

import jax
import jax.numpy as jnp
from jax import lax
from jax.experimental import pallas as pl
from jax.experimental.pallas import tpu as pltpu
from jax.experimental.pallas import tpu_sc as plsc

_NC = 2
_NS = 16
_NW = _NC * _NS
_NP = 10240


def _sc_segment_sums(p, ef, src, dst, zn, ze):
  N, D = p.shape
  E = src.shape[0] - 80
  DE = ef.shape[1]
  EPW = E // _NW
  B = 40
  NB = EPW // B
  CH = 40
  CPT = _NP // CH // _NS

  mesh = plsc.VectorSubcoreMesh(core_axis_name="c", subcore_axis_name="s")

  @functools.partial(
      pl.kernel,
      out_type=(
          jax.ShapeDtypeStruct((_NC * _NP, D), jnp.float32),
          jax.ShapeDtypeStruct((_NC * _NP, DE), jnp.float32),
      ),
      mesh=mesh,
      compiler_params=pltpu.CompilerParams(use_tc_tiling_on_sc=False),
      scratch_types=[
          pltpu.VMEM_SHARED((_NP, D), jnp.float32),
          pltpu.VMEM_SHARED((_NP, DE), jnp.float32),
          pltpu.VMEM((B,), jnp.int32),
          pltpu.VMEM((B,), jnp.int32),
          pltpu.VMEM((B,), jnp.int32),
          pltpu.VMEM((B,), jnp.int32),
          pltpu.VMEM((B, D), jnp.float32),
          pltpu.VMEM((B, D), jnp.float32),
          pltpu.VMEM((B, DE), jnp.float32),
          pltpu.VMEM((B, DE), jnp.float32),
          pltpu.SemaphoreType.DMA,
          pltpu.SemaphoreType.DMA,
          pltpu.SemaphoreType.DMA,
          pltpu.SemaphoreType.DMA,
          pltpu.SemaphoreType.DMA,
          pltpu.SemaphoreType.DMA,
          pltpu.SemaphoreType.DMA,
          pltpu.SemaphoreType.DMA,
          pltpu.SemaphoreType.DMA,
          pltpu.SemaphoreType.DMA,
          pltpu.SemaphoreType.DMA,
          pltpu.SemaphoreType.DMA,
      ],
  )
  def seg(p_hbm, ef_hbm, src_hbm, dst_hbm, zn_hbm, ze_hbm, outn, oute,
          aggn, agge, sidx0, sidx1, didx0, didx1, rows0, rows1,
          erows0, erows1, ss0, ss1, sd0, sd1, sg0, sg1, sq0, sq1,
          sn0, sn1, se0, se1):
    cid = lax.axis_index("c")
    sid = lax.axis_index("s")
    wid = sid * _NC + cid

    pltpu.sync_copy(zn_hbm, rows0)
    pltpu.sync_copy(ze_hbm, erows0)
    for i in range(CPT):
      r0 = pl.multiple_of((sid * CPT + i) * CH, CH)
      pltpu.sync_copy(rows0, aggn.at[pl.ds(r0, CH)])
      pltpu.sync_copy(erows0, agge.at[pl.ds(r0, CH)])

    plsc.subcore_barrier()

    e0 = wid * EPW
    sidx = (sidx0, sidx1)
    didx = (didx0, didx1)
    rows = (rows0, rows1)
    erows = (erows0, erows1)
    ss = (ss0, ss1)
    sd = (sd0, sd1)
    sg = (sg0, sg1)
    sq = (sq0, sq1)
    sn = (sn0, sn1)
    se = (se0, se1)

    def fire_sidx(j, b):
      pltpu.async_copy(src_hbm.at[pl.ds(e0 + j * B, B)], sidx[b], ss[b])

    def wait_sidx(b):
      pltpu.make_async_copy(src_hbm.at[pl.ds(0, B)], sidx[b], ss[b]).wait()

    def fire_didx(j, b):
      pltpu.async_copy(dst_hbm.at[pl.ds(e0 + j * B, B)], didx[b], sd[b])

    def wait_didx(b):
      pltpu.make_async_copy(dst_hbm.at[pl.ds(0, B)], didx[b], sd[b]).wait()

    def fire_gq(j, b):
      pltpu.async_copy(p_hbm.at[sidx[b]], rows[b], sg[b])
      pltpu.async_copy(ef_hbm.at[pl.ds(e0 + j * B, B)], erows[b], sq[b])

    def drain_gq(j, b):
      pltpu.make_async_copy(p_hbm.at[sidx[b]], rows[b], sg[b]).wait()
      pltpu.make_async_copy(ef_hbm.at[pl.ds(e0 + j * B, B)], erows[b],
                            sq[b]).wait()

    def fire_scatter(b):
      pltpu.async_copy(rows[b], aggn.at[didx[b]], sn[b], add=True)
      pltpu.async_copy(erows[b], agge.at[didx[b]], se[b], add=True)

    def wait_scatter(b):
      pltpu.make_async_copy(rows[b], aggn.at[didx[b]], sn[b]).wait()
      pltpu.make_async_copy(erows[b], agge.at[didx[b]], se[b]).wait()

    fire_sidx(0, 0)
    fire_sidx(1, 1)
    fire_didx(0, 0)
    wait_sidx(0)
    fire_gq(0, 0)
    fire_didx(1, 1)
    wait_sidx(1)
    fire_gq(1, 1)
    drain_gq(0, 0)
    fire_sidx(2, 0)
    wait_didx(0)
    fire_scatter(0)
    drain_gq(1, 1)
    fire_sidx(3, 1)
    wait_didx(1)
    fire_scatter(1)

    def body(i, carry):
      j = i * 2
      wait_scatter(0)
      fire_didx(j, 0)
      wait_sidx(0)
      fire_gq(j, 0)
      wait_scatter(1)
      fire_didx(j + 1, 1)
      wait_sidx(1)
      fire_gq(j + 1, 1)
      drain_gq(j, 0)
      fire_sidx(j + 2, 0)
      wait_didx(0)
      fire_scatter(0)
      drain_gq(j + 1, 1)
      fire_sidx(j + 3, 1)
      wait_didx(1)
      fire_scatter(1)
      return carry

    lax.fori_loop(1, NB // 2, body, 0)
    wait_scatter(0)
    wait_scatter(1)
    wait_sidx(0)
    wait_sidx(1)
    plsc.subcore_barrier()

    for i in range(CPT):
      r0 = pl.multiple_of((sid * CPT + i) * CH, CH)
      o0 = pl.multiple_of(cid * _NP + r0, CH)
      pltpu.sync_copy(aggn.at[pl.ds(r0, CH)], rows0)
      pltpu.sync_copy(rows0, outn.at[pl.ds(o0, CH)])
      pltpu.sync_copy(agge.at[pl.ds(r0, CH)], erows0)
      pltpu.sync_copy(erows0, oute.at[pl.ds(o0, CH)])

  return seg(p, ef, src, dst, zn, ze)


def _tc_matmul(x, wt):
  M, K = x.shape
  OUT = wt.shape[1]
  BM = 20000 if M % 20000 == 0 else 10000

  def body(x_ref, w_ref, out_ref):
    out_ref[...] = jnp.dot(x_ref[...], w_ref[...],
                           preferred_element_type=jnp.float32)

  return pl.pallas_call(
      body,
      grid=(M // BM,),
      in_specs=[
          pl.BlockSpec((BM, K), lambda i: (i, 0)),
          pl.BlockSpec((K, OUT), lambda i: (0, 0)),
      ],
      out_specs=pl.BlockSpec((BM, OUT), lambda i: (i, 0)),
      out_shape=jax.ShapeDtypeStruct((M, OUT), jnp.float32),
  )(x, wt)


def _tc_final(nf, an0, an1, ae0, ae1, w1t, w3t, b2):
  N, D = nf.shape
  DE = ae0.shape[1]
  OUT = w1t.shape[1]
  BM = 10000

  def body(nf_ref, an0_ref, an1_ref, ae0_ref, ae1_ref, w1_ref, w3_ref,
           b_ref, out_ref):
    acc = jnp.dot(nf_ref[...], w1_ref[...], preferred_element_type=jnp.float32)
    ae = ae0_ref[...] + ae1_ref[...]
    acc += jnp.dot(ae, w3_ref[...], preferred_element_type=jnp.float32)
    out_ref[...] = acc + an0_ref[...] + an1_ref[...] + b_ref[...]

  return pl.pallas_call(
      body,
      grid=(N // BM,),
      in_specs=[
          pl.BlockSpec((BM, D), lambda i: (i, 0)),
          pl.BlockSpec((BM, OUT), lambda i: (i, 0)),
          pl.BlockSpec((BM, OUT), lambda i: (i, 0)),
          pl.BlockSpec((BM, DE), lambda i: (i, 0)),
          pl.BlockSpec((BM, DE), lambda i: (i, 0)),
          pl.BlockSpec((D, OUT), lambda i: (0, 0)),
          pl.BlockSpec((DE, OUT), lambda i: (0, 0)),
          pl.BlockSpec((1, OUT), lambda i: (0, 0)),
      ],
      out_specs=pl.BlockSpec((BM, OUT), lambda i: (i, 0)),
      out_shape=jax.ShapeDtypeStruct((N, OUT), jnp.float32),
  )(nf, an0, an1, ae0, ae1, w1t, w3t, b2)


def kernel(node_features, edge_features, edge_index, W, b):
  N, D = node_features.shape
  DE = edge_features.shape[1]
  src = jnp.pad(edge_index[0], (0, 80))
  dst = edge_index[1]
  wt = W.T
  p = _tc_matmul(node_features, wt[D:2 * D])
  zn = jnp.zeros((40, wt.shape[1]), jnp.float32)
  ze = jnp.zeros((40, DE), jnp.float32)
  outn, oute = _sc_segment_sums(p, edge_features, src, dst, zn, ze)
  an0, an1 = outn[:N], outn[_NP:_NP + N]
  ae0, ae1 = oute[:N], oute[_NP:_NP + N]
  b2 = b.reshape(1, -1)
  return _tc_final(node_features, an0, an1, ae0, ae1, wt[:D], wt[2 * D:], b2)


# speedup vs baseline: 1.0945x; 1.0945x over previous
import functools

import jax
import jax.numpy as jnp
from jax import lax
from jax.experimental import pallas as pl
from jax.experimental.pallas import tpu as pltpu
from jax.experimental.pallas import tpu_sc as plsc

_NC = 2
_NS = 16
_NW = _NC * _NS
_NP = 10240


def _sc_segment_sums(p, ef, src, dst, zn, ze):
  N, D = p.shape
  E, DE = ef.shape
  EPW = E // _NW
  B = 80
  NB = EPW // B
  CH = 80
  CPT = _NP // CH // _NS

  mesh = plsc.VectorSubcoreMesh(core_axis_name="c", subcore_axis_name="s")

  @functools.partial(
      pl.kernel,
      out_type=(
          jax.ShapeDtypeStruct((_NC * _NP, D), jnp.float32),
          jax.ShapeDtypeStruct((_NC * _NP, DE), jnp.float32),
      ),
      mesh=mesh,
      compiler_params=pltpu.CompilerParams(use_tc_tiling_on_sc=False),
      scratch_types=[
          pltpu.VMEM_SHARED((_NP, D), jnp.float32),
          pltpu.VMEM_SHARED((_NP, DE), jnp.float32),
          pltpu.VMEM((B,), jnp.int32),
          pltpu.VMEM((B,), jnp.int32),
          pltpu.VMEM((B,), jnp.int32),
          pltpu.VMEM((B,), jnp.int32),
          pltpu.VMEM((B, D), jnp.float32),
          pltpu.VMEM((B, D), jnp.float32),
          pltpu.VMEM((B, DE), jnp.float32),
          pltpu.VMEM((B, DE), jnp.float32),
          pltpu.SemaphoreType.DMA,
          pltpu.SemaphoreType.DMA,
          pltpu.SemaphoreType.DMA,
          pltpu.SemaphoreType.DMA,
          pltpu.SemaphoreType.DMA,
          pltpu.SemaphoreType.DMA,
      ],
  )
  def seg(p_hbm, ef_hbm, src_hbm, dst_hbm, zn_hbm, ze_hbm, outn, oute,
          aggn, agge, sidx0, sidx1, didx0, didx1, rows0, rows1,
          erows0, erows1, si0, si1, sg0, sg1, sq0, sq1):
    cid = lax.axis_index("c")
    sid = lax.axis_index("s")
    wid = sid * _NC + cid

    pltpu.sync_copy(zn_hbm, rows0)
    pltpu.sync_copy(ze_hbm, erows0)
    for i in range(CPT):
      r0 = pl.multiple_of((sid * CPT + i) * CH, CH)
      pltpu.sync_copy(rows0, aggn.at[pl.ds(r0, CH)])
      pltpu.sync_copy(erows0, agge.at[pl.ds(r0, CH)])

    plsc.subcore_barrier()

    e0 = wid * EPW
    sidx = (sidx0, sidx1)
    didx = (didx0, didx1)
    rows = (rows0, rows1)
    erows = (erows0, erows1)
    si = (si0, si1)
    sg = (sg0, sg1)
    sq = (sq0, sq1)

    def fire_idx(j, b):
      base = e0 + j * B
      pltpu.async_copy(src_hbm.at[pl.ds(base, B)], sidx[b], si[b])
      pltpu.async_copy(dst_hbm.at[pl.ds(base, B)], didx[b], si[b])

    def wait_idx(b):
      pltpu.make_async_copy(src_hbm.at[pl.ds(0, B)], sidx[b], si[b]).wait()
      pltpu.make_async_copy(dst_hbm.at[pl.ds(0, B)], didx[b], si[b]).wait()

    def fire_gq(j, b):
      pltpu.async_copy(p_hbm.at[sidx[b]], rows[b], sg[b])
      pltpu.async_copy(ef_hbm.at[pl.ds(e0 + j * B, B)], erows[b], sq[b])

    def drain_scatter(j, b):
      pltpu.make_async_copy(p_hbm.at[sidx[b]], rows[b], sg[b]).wait()
      pltpu.make_async_copy(ef_hbm.at[pl.ds(e0 + j * B, B)], erows[b],
                            sq[b]).wait()
      pltpu.sync_copy(rows[b], aggn.at[didx[b]], add=True)
      pltpu.sync_copy(erows[b], agge.at[didx[b]], add=True)

    fire_idx(0, 0)
    fire_idx(1, 1)

    def body(i, carry):
      j = i * 2
      wait_idx(0)
      fire_gq(j, 0)
      wait_idx(1)
      fire_gq(j + 1, 1)
      drain_scatter(j, 0)
      fire_idx(j + 2, 0)
      drain_scatter(j + 1, 1)
      fire_idx(j + 3, 1)
      return carry

    lax.fori_loop(0, (NB - 3) // 2, body, 0)
    wait_idx(0)
    fire_gq(NB - 3, 0)
    wait_idx(1)
    fire_gq(NB - 2, 1)
    drain_scatter(NB - 3, 0)
    fire_idx(NB - 1, 0)
    drain_scatter(NB - 2, 1)
    wait_idx(0)
    fire_gq(NB - 1, 0)
    drain_scatter(NB - 1, 0)
    plsc.subcore_barrier()

    for i in range(CPT):
      r0 = pl.multiple_of((sid * CPT + i) * CH, CH)
      o0 = pl.multiple_of(cid * _NP + r0, CH)
      pltpu.sync_copy(aggn.at[pl.ds(r0, CH)], rows0)
      pltpu.sync_copy(rows0, outn.at[pl.ds(o0, CH)])
      pltpu.sync_copy(agge.at[pl.ds(r0, CH)], erows0)
      pltpu.sync_copy(erows0, oute.at[pl.ds(o0, CH)])

  return seg(p, ef, src, dst, zn, ze)


def _tc_matmul(x, wt):
  M, K = x.shape
  OUT = wt.shape[1]
  BM = 20000 if M % 20000 == 0 else 10000

  def body(x_ref, w_ref, out_ref):
    out_ref[...] = jnp.dot(x_ref[...], w_ref[...],
                           preferred_element_type=jnp.float32)

  return pl.pallas_call(
      body,
      grid=(M // BM,),
      in_specs=[
          pl.BlockSpec((BM, K), lambda i: (i, 0)),
          pl.BlockSpec((K, OUT), lambda i: (0, 0)),
      ],
      out_specs=pl.BlockSpec((BM, OUT), lambda i: (i, 0)),
      out_shape=jax.ShapeDtypeStruct((M, OUT), jnp.float32),
  )(x, wt)


def _tc_final(nf, an0, an1, ae0, ae1, w1t, w3t, b2):
  N, D = nf.shape
  DE = ae0.shape[1]
  OUT = w1t.shape[1]
  BM = 10000

  def body(nf_ref, an0_ref, an1_ref, ae0_ref, ae1_ref, w1_ref, w3_ref,
           b_ref, out_ref):
    acc = jnp.dot(nf_ref[...], w1_ref[...], preferred_element_type=jnp.float32)
    ae = ae0_ref[...] + ae1_ref[...]
    acc += jnp.dot(ae, w3_ref[...], preferred_element_type=jnp.float32)
    out_ref[...] = acc + an0_ref[...] + an1_ref[...] + b_ref[...]

  return pl.pallas_call(
      body,
      grid=(N // BM,),
      in_specs=[
          pl.BlockSpec((BM, D), lambda i: (i, 0)),
          pl.BlockSpec((BM, OUT), lambda i: (i, 0)),
          pl.BlockSpec((BM, OUT), lambda i: (i, 0)),
          pl.BlockSpec((BM, DE), lambda i: (i, 0)),
          pl.BlockSpec((BM, DE), lambda i: (i, 0)),
          pl.BlockSpec((D, OUT), lambda i: (0, 0)),
          pl.BlockSpec((DE, OUT), lambda i: (0, 0)),
          pl.BlockSpec((1, OUT), lambda i: (0, 0)),
      ],
      out_specs=pl.BlockSpec((BM, OUT), lambda i: (i, 0)),
      out_shape=jax.ShapeDtypeStruct((N, OUT), jnp.float32),
  )(nf, an0, an1, ae0, ae1, w1t, w3t, b2)


def kernel(node_features, edge_features, edge_index, W, b):
  N, D = node_features.shape
  DE = edge_features.shape[1]
  src = edge_index[0]
  dst = edge_index[1]
  wt = W.T
  p = _tc_matmul(node_features, wt[D:2 * D])
  zn = jnp.zeros((80, wt.shape[1]), jnp.float32)
  ze = jnp.zeros((80, DE), jnp.float32)
  outn, oute = _sc_segment_sums(p, edge_features, src, dst, zn, ze)
  an0, an1 = outn[:N], outn[_NP:_NP + N]
  ae0, ae1 = oute[:N], oute[_NP:_NP + N]
  b2 = b.reshape(1, -1)
  return _tc_final(node_features, an0, an1, ae0, ae1, wt[:D], wt[2 * D:], b2)
